# Initial kernel scaffold; baseline (speedup 1.0000x reference)
import jax
import jax.numpy as jnp
from jax.experimental import pallas as pl


def kernel(x, edge_index, W, b):
    raise NotImplementedError("write your pallas kernel here")


# speedup vs baseline: 27.2476x; 27.2476x over previous
import functools

import jax
import jax.numpy as jnp
from jax import lax
from jax.experimental import pallas as pl
from jax.experimental.pallas import tpu as pltpu
from jax.experimental.pallas import tpu_sc as plsc

N = 10000
E = 320000
D = 128
NP = 10240
NC, NS = 2, 16
NW = NC * NS
K = 80
EROWS = E // K
CPW = EROWS // NW
RPT = NP // NS
ZR = 80

_mesh = plsc.VectorSubcoreMesh(
    core_axis_name="c", subcore_axis_name="s", num_cores=NC, num_subcores=NS)


@functools.partial(
    pl.kernel,
    out_type=jax.ShapeDtypeStruct((NC, NP), jnp.float32),
    mesh=_mesh,
    scratch_types=[
        pltpu.VMEM((CPW, K), jnp.int32),
        pltpu.VMEM((K,), jnp.float32),
        pltpu.VMEM((RPT,), jnp.float32),
        pltpu.VMEM_SHARED((NP,), jnp.float32),
    ],
)
def _deg_kernel(dst_hbm, out_hbm, idx_v, ones_v, zb_v, deg_sh):
    c = lax.axis_index("c")
    s = lax.axis_index("s")
    wid = s * NC + c
    pltpu.sync_copy(dst_hbm.at[wid], idx_v)
    for k in range(K // 16):
        ones_v[pl.ds(k * 16, 16)] = jnp.ones((16,), jnp.float32)

    @pl.loop(0, RPT // 16)
    def _zero(i):
        zb_v[pl.ds(i * 16, 16)] = jnp.zeros((16,), jnp.float32)

    pltpu.sync_copy(zb_v, deg_sh.at[pl.ds(s * RPT, RPT)])
    plsc.subcore_barrier()

    @pl.loop(0, CPW)
    def _hist(j):
        pltpu.sync_copy(ones_v, deg_sh.at[idx_v.at[j]], add=True)

    plsc.subcore_barrier()
    pltpu.sync_copy(deg_sh.at[pl.ds(s * RPT, RPT)],
                    out_hbm.at[c, pl.ds(s * RPT, RPT)])


@functools.partial(
    pl.kernel,
    out_type=jax.ShapeDtypeStruct((NC, NP, D), jnp.float32),
    mesh=_mesh,
    scratch_types=[
        pltpu.VMEM((CPW, K), jnp.int32),
        pltpu.VMEM((CPW, K), jnp.int32),
        pltpu.VMEM((K, D), jnp.float32),
        pltpu.VMEM_SHARED((NP, D), jnp.float32),
        pltpu.SemaphoreType.DMA,
    ],
)
def _edge_kernel(y_hbm, src_hbm, dst_hbm, out_hbm,
                 si_v, di_v, rows_v, acc_sh, sem):
    c = lax.axis_index("c")
    s = lax.axis_index("s")
    wid = s * NC + c
    pltpu.sync_copy(src_hbm.at[wid], si_v)
    pltpu.sync_copy(dst_hbm.at[wid], di_v)

    @pl.loop(0, ZR)
    def _zero(r):
        for l in range(D // 16):
            rows_v[r, pl.ds(l * 16, 16)] = jnp.zeros((16,), jnp.float32)

    for t in range(RPT // ZR):
        pltpu.sync_copy(rows_v, acc_sh.at[pl.ds(s * RPT + t * ZR, ZR)])
    plsc.subcore_barrier()

    @pl.loop(0, CPW)
    def _edges(j):
        pltpu.async_copy(y_hbm.at[si_v.at[j]], rows_v, sem).wait()
        pltpu.sync_copy(rows_v, acc_sh.at[di_v.at[j]], add=True)

    plsc.subcore_barrier()
    pltpu.sync_copy(acc_sh.at[pl.ds(s * RPT, RPT)],
                    out_hbm.at[c, pl.ds(s * RPT, RPT)])


def _mm_body(xp_ref, w_ref, degr_ref, y_ref, dis_ref):
    deg = degr_ref[0] + degr_ref[1] + 1.0
    dis = lax.rsqrt(deg)
    dis_ref[...] = dis
    y_ref[...] = jnp.dot(xp_ref[...] * dis, w_ref[...],
                         preferred_element_type=jnp.float32)


def _comb_body(acc_ref, y_ref, dis_ref, b_ref, o_ref):
    o_ref[...] = dis_ref[...] * (acc_ref[0] + acc_ref[1] + y_ref[...]) \
        + b_ref[...]


def kernel(x, edge_index, W, b):
    src2d = edge_index[0].reshape(NW, CPW, K)
    dst2d = edge_index[1].reshape(NW, CPW, K)
    xp = jnp.pad(x, ((0, NP - N), (0, 0)))

    degparts = _deg_kernel(dst2d)

    y, dis = pl.pallas_call(
        _mm_body,
        out_shape=[jax.ShapeDtypeStruct((NP, D), jnp.float32),
                   jax.ShapeDtypeStruct((NP, 1), jnp.float32)],
    )(xp, W, degparts.reshape(NC, NP, 1))

    accparts = _edge_kernel(y, src2d, dst2d)

    outp = pl.pallas_call(
        _comb_body,
        out_shape=jax.ShapeDtypeStruct((NP, D), jnp.float32),
    )(accparts, y, dis, b.reshape(1, D))

    return outp[:N]


# speedup vs baseline: 41.2858x; 1.5152x over previous
import functools

import jax
import jax.numpy as jnp
from jax import lax
from jax.experimental import pallas as pl
from jax.experimental.pallas import tpu as pltpu
from jax.experimental.pallas import tpu_sc as plsc

N = 10000
E = 320000
D = 128
NP = 10240
NC, NS = 2, 16
NW = NC * NS
EPW = E // NW
K = 128
CPW = 80
PAD = CPW * K - EPW
PH = 5
PC = CPW // PH
RPT = NP // NS

_mesh = plsc.VectorSubcoreMesh(
    core_axis_name="c", subcore_axis_name="s", num_cores=NC, num_subcores=NS)


@functools.partial(
    pl.kernel,
    out_type=jax.ShapeDtypeStruct((NC, NP), jnp.float32),
    mesh=_mesh,
    scratch_types=[
        pltpu.VMEM((CPW, K), jnp.int32),
        pltpu.VMEM((K,), jnp.float32),
        pltpu.VMEM((RPT,), jnp.float32),
        pltpu.VMEM_SHARED((NP,), jnp.float32),
    ],
)
def _deg_kernel(dst_hbm, out_hbm, idx_v, ones_v, zb_v, deg_sh):
    c = lax.axis_index("c")
    s = lax.axis_index("s")
    wid = s * NC + c
    pltpu.sync_copy(dst_hbm.at[wid], idx_v)
    for k in range(K // 16):
        ones_v[pl.ds(k * 16, 16)] = jnp.ones((16,), jnp.float32)

    @pl.loop(0, RPT // 16)
    def _zero(i):
        zb_v[pl.ds(i * 16, 16)] = jnp.zeros((16,), jnp.float32)

    pltpu.sync_copy(zb_v, deg_sh.at[pl.ds(s * RPT, RPT)])
    plsc.subcore_barrier()

    @pl.loop(0, CPW)
    def _hist(j):
        pltpu.sync_copy(ones_v, deg_sh.at[idx_v.at[j]], add=True)

    plsc.subcore_barrier()
    pltpu.sync_copy(deg_sh.at[pl.ds(s * RPT, RPT)],
                    out_hbm.at[c, pl.ds(s * RPT, RPT)])


@functools.partial(
    pl.kernel,
    out_type=jax.ShapeDtypeStruct((NC, NP, D), jnp.float32),
    mesh=_mesh,
    scratch_types=[
        pltpu.VMEM((PC, K), jnp.int32),
        pltpu.VMEM((PC, K), jnp.int32),
        pltpu.VMEM((K, D), jnp.float32),
        pltpu.VMEM((K, D), jnp.float32),
        pltpu.VMEM_SHARED((NP, D), jnp.float32),
        pltpu.SemaphoreType.DMA,
        pltpu.SemaphoreType.DMA,
    ],
)
def _edge_kernel(y_hbm, src_hbm, dst_hbm, out_hbm,
                 si_v, di_v, rows0_v, rows1_v, acc_sh, sem0, sem1):
    c = lax.axis_index("c")
    s = lax.axis_index("s")
    wid = s * NC + c

    @pl.loop(0, K)
    def _zero(r):
        for l in range(D // 16):
            rows0_v[r, pl.ds(l * 16, 16)] = jnp.zeros((16,), jnp.float32)

    for t in range(RPT // K):
        pltpu.sync_copy(rows0_v, acc_sh.at[pl.ds(s * RPT + t * K, K)])
    plsc.subcore_barrier()

    def _gather(j, buf, sem):
        pltpu.async_copy(y_hbm.at[si_v.at[j]], buf, sem)

    def _drain_scatter(j, buf, sem):
        pltpu.make_async_copy(y_hbm.at[si_v.at[j]], buf, sem).wait()
        pltpu.sync_copy(buf, acc_sh.at[di_v.at[j]], add=True)

    for p in range(PH):
        pltpu.sync_copy(src_hbm.at[wid, pl.ds(p * PC, PC)], si_v)
        pltpu.sync_copy(dst_hbm.at[wid, pl.ds(p * PC, PC)], di_v)
        _gather(0, rows0_v, sem0)

        @pl.loop(0, (PC - 2) // 2)
        def _edges(i):
            a = 2 * i
            _gather(a + 1, rows1_v, sem1)
            _drain_scatter(a, rows0_v, sem0)
            _gather(a + 2, rows0_v, sem0)
            _drain_scatter(a + 1, rows1_v, sem1)

        _gather(PC - 1, rows1_v, sem1)
        _drain_scatter(PC - 2, rows0_v, sem0)
        _drain_scatter(PC - 1, rows1_v, sem1)

    plsc.subcore_barrier()
    pltpu.sync_copy(acc_sh.at[pl.ds(s * RPT, RPT)],
                    out_hbm.at[c, pl.ds(s * RPT, RPT)])


def _mm_body(x_ref, w_ref, degr_ref, y_ref, dis_ref):
    deg = degr_ref[0, :N] + degr_ref[1, :N] + 1.0
    dis = lax.rsqrt(deg)
    dis_ref[...] = dis
    y_ref[...] = jnp.dot(x_ref[...] * dis, w_ref[...],
                         preferred_element_type=jnp.float32)


def _comb_body(acc_ref, y_ref, dis_ref, b_ref, o_ref):
    o_ref[...] = dis_ref[...] * (
        acc_ref[0, :N] + acc_ref[1, :N] + y_ref[...]) + b_ref[...]


def _pad_edges(row, pad_vals):
    pads = jnp.broadcast_to(pad_vals[None, :], (NW, PAD))
    return jnp.concatenate([row.reshape(NW, EPW), pads], axis=1) \
        .reshape(NW, CPW, K)


def kernel(x, edge_index, W, b):
    ar = jnp.arange(PAD, dtype=jnp.int32)
    src3d = _pad_edges(edge_index[0], ar % 1024)
    dst3d = _pad_edges(edge_index[1], N + (ar % (NP - N)))

    degparts = _deg_kernel(dst3d)

    y, dis = pl.pallas_call(
        _mm_body,
        out_shape=[jax.ShapeDtypeStruct((N, D), jnp.float32),
                   jax.ShapeDtypeStruct((N, 1), jnp.float32)],
    )(x, W, degparts.reshape(NC, NP, 1))

    accparts = _edge_kernel(y, src3d, dst3d)

    out = pl.pallas_call(
        _comb_body,
        out_shape=jax.ShapeDtypeStruct((N, D), jnp.float32),
    )(accparts, y, dis, b.reshape(1, D))

    return out


# speedup vs baseline: 43.6850x; 1.0581x over previous
import functools

import jax
import jax.numpy as jnp
from jax import lax
from jax.experimental import pallas as pl
from jax.experimental.pallas import tpu as pltpu
from jax.experimental.pallas import tpu_sc as plsc

N = 10000
E = 320000
D = 128
NP = 10240
NC, NS = 2, 16
NW = NC * NS
EPW = E // NW
K = 128
CPW = 80
PAD = CPW * K - EPW
PH = 5
PC = CPW // PH
RPT = NP // NS

_mesh = plsc.VectorSubcoreMesh(
    core_axis_name="c", subcore_axis_name="s", num_cores=NC, num_subcores=NS)


@functools.partial(
    pl.kernel,
    out_type=jax.ShapeDtypeStruct((NC, NP), jnp.float32),
    mesh=_mesh,
    scratch_types=[
        pltpu.VMEM((CPW, K), jnp.int32),
        pltpu.VMEM((K,), jnp.float32),
        pltpu.VMEM((RPT,), jnp.float32),
        pltpu.VMEM_SHARED((NP,), jnp.float32),
    ],
)
def _deg_kernel(dst_hbm, out_hbm, idx_v, ones_v, zb_v, deg_sh):
    c = lax.axis_index("c")
    s = lax.axis_index("s")
    wid = s * NC + c
    pltpu.sync_copy(dst_hbm.at[wid], idx_v)
    for k in range(K // 16):
        ones_v[pl.ds(k * 16, 16)] = jnp.ones((16,), jnp.float32)

    @pl.loop(0, RPT // 16)
    def _zero(i):
        zb_v[pl.ds(i * 16, 16)] = jnp.zeros((16,), jnp.float32)

    pltpu.sync_copy(zb_v, deg_sh.at[pl.ds(s * RPT, RPT)])
    plsc.subcore_barrier()

    @pl.loop(0, CPW)
    def _hist(j):
        pltpu.sync_copy(ones_v, deg_sh.at[idx_v.at[j]], add=True)

    plsc.subcore_barrier()
    pltpu.sync_copy(deg_sh.at[pl.ds(s * RPT, RPT)],
                    out_hbm.at[c, pl.ds(s * RPT, RPT)])


@functools.partial(
    pl.kernel,
    out_type=jax.ShapeDtypeStruct((NC, NP, D), jnp.float32),
    mesh=_mesh,
    scratch_types=[
        pltpu.VMEM((PC, K), jnp.int32),
        pltpu.VMEM((PC, K), jnp.int32),
        pltpu.VMEM((PC, K), jnp.int32),
        pltpu.VMEM((PC, K), jnp.int32),
        pltpu.VMEM((K, D), jnp.float32),
        pltpu.VMEM((K, D), jnp.float32),
        pltpu.VMEM_SHARED((NP, D), jnp.float32),
        pltpu.SemaphoreType.DMA,
        pltpu.SemaphoreType.DMA,
        pltpu.SemaphoreType.DMA,
    ],
)
def _edge_kernel(y_hbm, src_hbm, dst_hbm, out_hbm,
                 siA_v, diA_v, siB_v, diB_v, rows0_v, rows1_v, acc_sh,
                 sem0, sem1, semi):
    c = lax.axis_index("c")
    s = lax.axis_index("s")
    wid = s * NC + c

    @pl.loop(0, K)
    def _zero(r):
        for l in range(D // 16):
            rows0_v[r, pl.ds(l * 16, 16)] = jnp.zeros((16,), jnp.float32)

    for t in range(RPT // K):
        pltpu.sync_copy(rows0_v, acc_sh.at[pl.ds(s * RPT + t * K, K)])
    plsc.subcore_barrier()

    def _gather(si, j, buf, sem):
        pltpu.async_copy(y_hbm.at[si.at[j]], buf, sem)

    def _drain_scatter(si, di, j, buf, sem):
        pltpu.make_async_copy(y_hbm.at[si.at[j]], buf, sem).wait()
        pltpu.sync_copy(buf, acc_sh.at[di.at[j]], add=True)

    def _idx_load(p, si, di, sem):
        pltpu.async_copy(src_hbm.at[wid, pl.ds(p * PC, PC)], si, sem)
        pltpu.async_copy(dst_hbm.at[wid, pl.ds(p * PC, PC)], di, sem)

    def _idx_wait(p, si, di, sem):
        pltpu.make_async_copy(src_hbm.at[wid, pl.ds(p * PC, PC)], si,
                              sem).wait()
        pltpu.make_async_copy(dst_hbm.at[wid, pl.ds(p * PC, PC)], di,
                              sem).wait()

    sets = [(siA_v, diA_v), (siB_v, diB_v)]
    _idx_load(0, *sets[0], semi)
    _idx_wait(0, *sets[0], semi)
    if PH > 1:
        _idx_load(1, *sets[1], semi)
    _gather(sets[0][0], 0, rows0_v, sem0)

    for p in range(PH):
        si, di = sets[p % 2]

        @pl.loop(0, (PC - 2) // 2)
        def _edges(i):
            a = 2 * i
            _gather(si, a + 1, rows1_v, sem1)
            _drain_scatter(si, di, a, rows0_v, sem0)
            _gather(si, a + 2, rows0_v, sem0)
            _drain_scatter(si, di, a + 1, rows1_v, sem1)

        _gather(si, PC - 1, rows1_v, sem1)
        _drain_scatter(si, di, PC - 2, rows0_v, sem0)
        if p + 1 < PH:
            nsi, ndi = sets[(p + 1) % 2]
            _idx_wait(p + 1, nsi, ndi, semi)
            _gather(nsi, 0, rows0_v, sem0)
            _drain_scatter(si, di, PC - 1, rows1_v, sem1)
            if p + 2 < PH:
                _idx_load(p + 2, si, di, semi)
        else:
            _drain_scatter(si, di, PC - 1, rows1_v, sem1)

    plsc.subcore_barrier()
    pltpu.sync_copy(acc_sh.at[pl.ds(s * RPT, RPT)],
                    out_hbm.at[c, pl.ds(s * RPT, RPT)])


def _mm_body(x_ref, w_ref, degr_ref, y_ref, dis_ref):
    deg = degr_ref[0, :N] + degr_ref[1, :N] + 1.0
    dis = lax.rsqrt(deg)
    dis_ref[...] = dis
    y_ref[...] = jnp.dot(x_ref[...] * dis, w_ref[...],
                         preferred_element_type=jnp.float32)


def _comb_body(acc_ref, y_ref, dis_ref, b_ref, o_ref):
    o_ref[...] = dis_ref[...] * (
        acc_ref[0, :N] + acc_ref[1, :N] + y_ref[...]) + b_ref[...]


def _pad_edges(row, pad_vals):
    pads = jnp.broadcast_to(pad_vals[None, :], (NW, PAD))
    return jnp.concatenate([row.reshape(NW, EPW), pads], axis=1) \
        .reshape(NW, CPW, K)


def kernel(x, edge_index, W, b):
    ar = jnp.arange(PAD, dtype=jnp.int32)
    src3d = _pad_edges(edge_index[0], ar % 1024)
    dst3d = _pad_edges(edge_index[1], N + (ar % (NP - N)))

    degparts = _deg_kernel(dst3d)

    y, dis = pl.pallas_call(
        _mm_body,
        out_shape=[jax.ShapeDtypeStruct((N, D), jnp.float32),
                   jax.ShapeDtypeStruct((N, 1), jnp.float32)],
    )(x, W, degparts.reshape(NC, NP, 1))

    accparts = _edge_kernel(y, src3d, dst3d)

    out = pl.pallas_call(
        _comb_body,
        out_shape=jax.ShapeDtypeStruct((N, D), jnp.float32),
    )(accparts, y, dis, b.reshape(1, D))

    return out


# speedup vs baseline: 43.7282x; 1.0010x over previous
import functools

import jax
import jax.numpy as jnp
from jax import lax
from jax.experimental import pallas as pl
from jax.experimental.pallas import tpu as pltpu
from jax.experimental.pallas import tpu_sc as plsc

N = 10000
E = 320000
D = 128
NP = 10240
NC, NS = 2, 16
NW = NC * NS
EPW = E // NW
K = 128
CPW = 80
PAD = CPW * K - EPW
PH = 5
PC = CPW // PH
RPT = NP // NS

_mesh = plsc.VectorSubcoreMesh(
    core_axis_name="c", subcore_axis_name="s", num_cores=NC, num_subcores=NS)


@functools.partial(
    pl.kernel,
    out_type=jax.ShapeDtypeStruct((NC, NP), jnp.float32),
    mesh=_mesh,
    scratch_types=[
        pltpu.VMEM((CPW, K), jnp.int32),
        pltpu.VMEM((K,), jnp.float32),
        pltpu.VMEM((RPT,), jnp.float32),
        pltpu.VMEM_SHARED((NP,), jnp.float32),
    ],
)
def _deg_kernel(dst_hbm, out_hbm, idx_v, ones_v, zb_v, deg_sh):
    c = lax.axis_index("c")
    s = lax.axis_index("s")
    wid = s * NC + c
    pltpu.sync_copy(dst_hbm.at[wid], idx_v)
    for k in range(K // 16):
        ones_v[pl.ds(k * 16, 16)] = jnp.ones((16,), jnp.float32)

    @pl.loop(0, RPT // 16)
    def _zero(i):
        zb_v[pl.ds(i * 16, 16)] = jnp.zeros((16,), jnp.float32)

    pltpu.sync_copy(zb_v, deg_sh.at[pl.ds(s * RPT, RPT)])
    plsc.subcore_barrier()

    @pl.loop(0, CPW)
    def _hist(j):
        pltpu.sync_copy(ones_v, deg_sh.at[idx_v.at[j]], add=True)

    plsc.subcore_barrier()
    pltpu.sync_copy(deg_sh.at[pl.ds(s * RPT, RPT)],
                    out_hbm.at[c, pl.ds(s * RPT, RPT)])


@functools.partial(
    pl.kernel,
    out_type=jax.ShapeDtypeStruct((NC, NP, D), jnp.float32),
    mesh=_mesh,
    scratch_types=[
        pltpu.VMEM((PC, K), jnp.int32),
        pltpu.VMEM((PC, K), jnp.int32),
        pltpu.VMEM((PC, K), jnp.int32),
        pltpu.VMEM((PC, K), jnp.int32),
        pltpu.VMEM((K, D), jnp.float32),
        pltpu.VMEM((K, D), jnp.float32),
        pltpu.VMEM_SHARED((NP, D), jnp.float32),
        pltpu.SemaphoreType.DMA,
        pltpu.SemaphoreType.DMA,
        pltpu.SemaphoreType.DMA,
    ],
)
def _edge_kernel(y_hbm, src_hbm, dst_hbm, out_hbm,
                 siA_v, diA_v, siB_v, diB_v, rows0_v, rows1_v, acc_sh,
                 sem0, sem1, semi):
    c = lax.axis_index("c")
    s = lax.axis_index("s")
    wid = s * NC + c

    @pl.loop(0, K)
    def _zero(r):
        for l in range(D // 16):
            rows0_v[r, pl.ds(l * 16, 16)] = jnp.zeros((16,), jnp.float32)

    for t in range(RPT // K):
        pltpu.sync_copy(rows0_v, acc_sh.at[pl.ds(s * RPT + t * K, K)])
    plsc.subcore_barrier()

    def _gather(si, j, buf, sem):
        pltpu.async_copy(y_hbm.at[si.at[j, pl.ds(0, K // 2)]],
                         buf.at[pl.ds(0, K // 2)], sem)
        pltpu.async_copy(y_hbm.at[si.at[j, pl.ds(K // 2, K // 2)]],
                         buf.at[pl.ds(K // 2, K // 2)], sem)

    def _drain_scatter(si, di, j, buf, sem):
        pltpu.make_async_copy(y_hbm.at[si.at[j]], buf, sem).wait()
        pltpu.sync_copy(buf, acc_sh.at[di.at[j]], add=True)

    def _idx_load(p, si, di, sem):
        pltpu.async_copy(src_hbm.at[wid, pl.ds(p * PC, PC)], si, sem)
        pltpu.async_copy(dst_hbm.at[wid, pl.ds(p * PC, PC)], di, sem)

    def _idx_wait(p, si, di, sem):
        pltpu.make_async_copy(src_hbm.at[wid, pl.ds(p * PC, PC)], si,
                              sem).wait()
        pltpu.make_async_copy(dst_hbm.at[wid, pl.ds(p * PC, PC)], di,
                              sem).wait()

    sets = [(siA_v, diA_v), (siB_v, diB_v)]
    _idx_load(0, *sets[0], semi)
    _idx_wait(0, *sets[0], semi)
    if PH > 1:
        _idx_load(1, *sets[1], semi)
    _gather(sets[0][0], 0, rows0_v, sem0)

    for p in range(PH):
        si, di = sets[p % 2]

        @pl.loop(0, (PC - 2) // 2)
        def _edges(i):
            a = 2 * i
            _gather(si, a + 1, rows1_v, sem1)
            _drain_scatter(si, di, a, rows0_v, sem0)
            _gather(si, a + 2, rows0_v, sem0)
            _drain_scatter(si, di, a + 1, rows1_v, sem1)

        _gather(si, PC - 1, rows1_v, sem1)
        _drain_scatter(si, di, PC - 2, rows0_v, sem0)
        if p + 1 < PH:
            nsi, ndi = sets[(p + 1) % 2]
            _idx_wait(p + 1, nsi, ndi, semi)
            _gather(nsi, 0, rows0_v, sem0)
            _drain_scatter(si, di, PC - 1, rows1_v, sem1)
            if p + 2 < PH:
                _idx_load(p + 2, si, di, semi)
        else:
            _drain_scatter(si, di, PC - 1, rows1_v, sem1)

    plsc.subcore_barrier()
    pltpu.sync_copy(acc_sh.at[pl.ds(s * RPT, RPT)],
                    out_hbm.at[c, pl.ds(s * RPT, RPT)])


def _mm_body(x_ref, w_ref, degr_ref, y_ref, dis_ref):
    deg = degr_ref[0, :N] + degr_ref[1, :N] + 1.0
    dis = lax.rsqrt(deg)
    dis_ref[...] = dis
    y_ref[...] = jnp.dot(x_ref[...] * dis, w_ref[...],
                         preferred_element_type=jnp.float32)


def _comb_body(acc_ref, y_ref, dis_ref, b_ref, o_ref):
    o_ref[...] = dis_ref[...] * (
        acc_ref[0, :N] + acc_ref[1, :N] + y_ref[...]) + b_ref[...]


def _pad_edges(row, pad_vals):
    pads = jnp.broadcast_to(pad_vals[None, :], (NW, PAD))
    return jnp.concatenate([row.reshape(NW, EPW), pads], axis=1) \
        .reshape(NW, CPW, K)


def kernel(x, edge_index, W, b):
    ar = jnp.arange(PAD, dtype=jnp.int32)
    src3d = _pad_edges(edge_index[0], ar % 1024)
    dst3d = _pad_edges(edge_index[1], N + (ar % (NP - N)))

    degparts = _deg_kernel(dst3d)

    y, dis = pl.pallas_call(
        _mm_body,
        out_shape=[jax.ShapeDtypeStruct((N, D), jnp.float32),
                   jax.ShapeDtypeStruct((N, 1), jnp.float32)],
    )(x, W, degparts.reshape(NC, NP, 1))

    accparts = _edge_kernel(y, src3d, dst3d)

    out = pl.pallas_call(
        _comb_body,
        out_shape=jax.ShapeDtypeStruct((N, D), jnp.float32),
    )(accparts, y, dis, b.reshape(1, D))

    return out


# speedup vs baseline: 49.5560x; 1.1333x over previous
import functools

import jax
import jax.numpy as jnp
from jax import lax
from jax.experimental import pallas as pl
from jax.experimental.pallas import tpu as pltpu
from jax.experimental.pallas import tpu_sc as plsc

N = 10000
E = 320000
D = 128
NP = 10240
NC, NS = 2, 16
NW = NC * NS
EPW = E // NW
K = 128
CPW = 80
PAD = CPW * K - EPW
PH = 5
PC = CPW // PH
RPT = NP // NS

_mesh = plsc.VectorSubcoreMesh(
    core_axis_name="c", subcore_axis_name="s", num_cores=NC, num_subcores=NS)


@functools.partial(
    pl.kernel,
    out_type=jax.ShapeDtypeStruct((NC, NP), jnp.float32),
    mesh=_mesh,
    scratch_types=[
        pltpu.VMEM((CPW, K), jnp.int32),
        pltpu.VMEM((K,), jnp.float32),
        pltpu.VMEM((RPT,), jnp.float32),
        pltpu.VMEM_SHARED((NP,), jnp.float32),
    ],
)
def _deg_kernel(dst_hbm, out_hbm, idx_v, ones_v, zb_v, deg_sh):
    c = lax.axis_index("c")
    s = lax.axis_index("s")
    wid = s * NC + c
    pltpu.sync_copy(dst_hbm.at[wid], idx_v)
    for k in range(K // 16):
        ones_v[pl.ds(k * 16, 16)] = jnp.ones((16,), jnp.float32)

    @pl.loop(0, RPT // 16)
    def _zero(i):
        zb_v[pl.ds(i * 16, 16)] = jnp.zeros((16,), jnp.float32)

    pltpu.sync_copy(zb_v, deg_sh.at[pl.ds(s * RPT, RPT)])
    plsc.subcore_barrier()

    @pl.loop(0, CPW)
    def _hist(j):
        pltpu.sync_copy(ones_v, deg_sh.at[idx_v.at[j]], add=True)

    plsc.subcore_barrier()
    pltpu.sync_copy(deg_sh.at[pl.ds(s * RPT, RPT)],
                    out_hbm.at[c, pl.ds(s * RPT, RPT)])


@functools.partial(
    pl.kernel,
    out_type=jax.ShapeDtypeStruct((NC, NP, D), jnp.float32),
    mesh=_mesh,
    scratch_types=[
        pltpu.VMEM((PC, K), jnp.int32),
        pltpu.VMEM((PC, K), jnp.int32),
        pltpu.VMEM((PC, K), jnp.int32),
        pltpu.VMEM((PC, K), jnp.int32),
        pltpu.VMEM((K, D), jnp.float32),
        pltpu.VMEM((K, D), jnp.float32),
        pltpu.VMEM_SHARED((NP, D), jnp.float32),
        pltpu.SemaphoreType.DMA,
        pltpu.SemaphoreType.DMA,
        pltpu.SemaphoreType.DMA,
    ],
)
def _edge_kernel(y_hbm, src_hbm, dst_hbm, out_hbm,
                 siA_v, diA_v, siB_v, diB_v, rows0_v, rows1_v, acc_sh,
                 sem0, sem1, semi):
    c = lax.axis_index("c")
    s = lax.axis_index("s")
    wid = s * NC + c

    @pl.loop(0, K)
    def _zero(r):
        for l in range(D // 16):
            rows0_v[r, pl.ds(l * 16, 16)] = jnp.zeros((16,), jnp.float32)

    for t in range(RPT // K):
        pltpu.sync_copy(rows0_v, acc_sh.at[pl.ds(s * RPT + t * K, K)])
    plsc.subcore_barrier()

    def _gather(si, j, buf, sem):
        pltpu.async_copy(y_hbm.at[si.at[j]], buf, sem)

    def _drain_scatter(si, di, j, buf, sem):
        pltpu.make_async_copy(y_hbm.at[si.at[j]], buf, sem).wait()
        pltpu.sync_copy(buf, acc_sh.at[di.at[j]], add=True)

    def _idx_load(p, si, di, sem):
        pltpu.async_copy(src_hbm.at[wid, pl.ds(p * PC, PC)], si, sem)
        pltpu.async_copy(dst_hbm.at[wid, pl.ds(p * PC, PC)], di, sem)

    def _idx_wait(p, si, di, sem):
        pltpu.make_async_copy(src_hbm.at[wid, pl.ds(p * PC, PC)], si,
                              sem).wait()
        pltpu.make_async_copy(dst_hbm.at[wid, pl.ds(p * PC, PC)], di,
                              sem).wait()

    sets = [(siA_v, diA_v), (siB_v, diB_v)]
    _idx_load(0, *sets[0], semi)
    _idx_wait(0, *sets[0], semi)
    if PH > 1:
        _idx_load(1, *sets[1], semi)
    _gather(sets[0][0], 0, rows0_v, sem0)

    for p in range(PH):
        si, di = sets[p % 2]

        @pl.loop(0, (PC - 2) // 2)
        def _edges(i):
            a = 2 * i
            _gather(si, a + 1, rows1_v, sem1)
            _drain_scatter(si, di, a, rows0_v, sem0)
            _gather(si, a + 2, rows0_v, sem0)
            _drain_scatter(si, di, a + 1, rows1_v, sem1)

        _gather(si, PC - 1, rows1_v, sem1)
        _drain_scatter(si, di, PC - 2, rows0_v, sem0)
        if p + 1 < PH:
            nsi, ndi = sets[(p + 1) % 2]
            _idx_wait(p + 1, nsi, ndi, semi)
            _gather(nsi, 0, rows0_v, sem0)
            _drain_scatter(si, di, PC - 1, rows1_v, sem1)
            if p + 2 < PH:
                _idx_load(p + 2, si, di, semi)
        else:
            _drain_scatter(si, di, PC - 1, rows1_v, sem1)

    plsc.subcore_barrier()
    pltpu.sync_copy(acc_sh.at[pl.ds(s * RPT, RPT)],
                    out_hbm.at[c, pl.ds(s * RPT, RPT)])


def _dis_column(degr_ref):
    deg = degr_ref[0] + degr_ref[1] + 1.0
    return jnp.reshape(lax.rsqrt(deg), (NP, 1))[:N]


def _mm_body(x_ref, w_ref, degr_ref, y_ref):
    y_ref[...] = jnp.dot(x_ref[...] * _dis_column(degr_ref), w_ref[...],
                         preferred_element_type=jnp.float32)


def _comb_body(acc_ref, y_ref, degr_ref, b_ref, o_ref):
    o_ref[...] = _dis_column(degr_ref) * (
        acc_ref[0, :N] + acc_ref[1, :N] + y_ref[...]) + b_ref[...]


EROWS = E // K
PROWS = NW * CPW - EROWS


def kernel(x, edge_index, W, b):
    ei3 = edge_index.reshape(2, EROWS, K)
    ar = jnp.arange(PROWS * K, dtype=jnp.int32).reshape(PROWS, K)
    src3d = jnp.concatenate([ei3[0], ar % 1024]).reshape(NW, CPW, K)
    dst3d = jnp.concatenate([ei3[1], N + ar % (NP - N)]).reshape(NW, CPW, K)

    degparts = _deg_kernel(dst3d)

    y = pl.pallas_call(
        _mm_body,
        out_shape=jax.ShapeDtypeStruct((N, D), jnp.float32),
    )(x, W, degparts)

    accparts = _edge_kernel(y, src3d, dst3d)

    out = pl.pallas_call(
        _comb_body,
        out_shape=jax.ShapeDtypeStruct((N, D), jnp.float32),
    )(accparts, y, degparts, b.reshape(1, D))

    return out


# speedup vs baseline: 50.5234x; 1.0195x over previous
import functools

import jax
import jax.numpy as jnp
from jax import lax
from jax.experimental import pallas as pl
from jax.experimental.pallas import tpu as pltpu
from jax.experimental.pallas import tpu_sc as plsc

N = 10000
E = 320000
D = 128
NP = 10240
NC, NS = 2, 16
NW = NC * NS
EPW = E // NW
K = 128
CPW = 80
PAD = CPW * K - EPW
PH = 5
PC = CPW // PH
RPT = NP // NS

_mesh = plsc.VectorSubcoreMesh(
    core_axis_name="c", subcore_axis_name="s", num_cores=NC, num_subcores=NS)


@functools.partial(
    pl.kernel,
    out_type=jax.ShapeDtypeStruct((NC, NP), jnp.float32),
    mesh=_mesh,
    scratch_types=[
        pltpu.VMEM((CPW, K), jnp.int32),
        pltpu.VMEM((K,), jnp.float32),
        pltpu.VMEM((RPT,), jnp.float32),
        pltpu.VMEM_SHARED((NP,), jnp.float32),
    ],
)
def _deg_kernel(ei_hbm, out_hbm, idx_v, ones_v, zb_v, deg_sh):
    c = lax.axis_index("c")
    s = lax.axis_index("s")
    wid = s * NC + c
    pltpu.sync_copy(ei_hbm.at[1, pl.ds(wid * CPW, CPW)], idx_v)
    for k in range(K // 16):
        ones_v[pl.ds(k * 16, 16)] = jnp.ones((16,), jnp.float32)

    @pl.loop(0, RPT // 16)
    def _zero(i):
        zb_v[pl.ds(i * 16, 16)] = jnp.zeros((16,), jnp.float32)

    pltpu.sync_copy(zb_v, deg_sh.at[pl.ds(s * RPT, RPT)])
    plsc.subcore_barrier()

    @pl.loop(0, CPW)
    def _hist(j):
        pltpu.sync_copy(ones_v, deg_sh.at[idx_v.at[j]], add=True)

    plsc.subcore_barrier()
    pltpu.sync_copy(deg_sh.at[pl.ds(s * RPT, RPT)],
                    out_hbm.at[c, pl.ds(s * RPT, RPT)])


@functools.partial(
    pl.kernel,
    out_type=jax.ShapeDtypeStruct((NC, NP, D), jnp.float32),
    mesh=_mesh,
    scratch_types=[
        pltpu.VMEM((PC, K), jnp.int32),
        pltpu.VMEM((PC, K), jnp.int32),
        pltpu.VMEM((PC, K), jnp.int32),
        pltpu.VMEM((PC, K), jnp.int32),
        pltpu.VMEM((K, D), jnp.float32),
        pltpu.VMEM((K, D), jnp.float32),
        pltpu.VMEM_SHARED((NP, D), jnp.float32),
        pltpu.SemaphoreType.DMA,
        pltpu.SemaphoreType.DMA,
        pltpu.SemaphoreType.DMA,
    ],
)
def _edge_kernel(y_hbm, ei_hbm, out_hbm,
                 siA_v, diA_v, siB_v, diB_v, rows0_v, rows1_v, acc_sh,
                 sem0, sem1, semi):
    c = lax.axis_index("c")
    s = lax.axis_index("s")
    wid = s * NC + c

    @pl.loop(0, K)
    def _zero(r):
        for l in range(D // 16):
            rows0_v[r, pl.ds(l * 16, 16)] = jnp.zeros((16,), jnp.float32)

    for t in range(RPT // K):
        pltpu.sync_copy(rows0_v, acc_sh.at[pl.ds(s * RPT + t * K, K)])
    plsc.subcore_barrier()

    def _gather(si, j, buf, sem):
        pltpu.async_copy(y_hbm.at[si.at[j]], buf, sem)

    def _drain_scatter(si, di, j, buf, sem):
        pltpu.make_async_copy(y_hbm.at[si.at[j]], buf, sem).wait()
        pltpu.sync_copy(buf, acc_sh.at[di.at[j]], add=True)

    def _idx_load(p, si, di, sem):
        off = wid * CPW + p * PC
        pltpu.async_copy(ei_hbm.at[0, pl.ds(off, PC)], si, sem)
        pltpu.async_copy(ei_hbm.at[1, pl.ds(off, PC)], di, sem)

    def _idx_wait(p, si, di, sem):
        off = wid * CPW + p * PC
        pltpu.make_async_copy(ei_hbm.at[0, pl.ds(off, PC)], si, sem).wait()
        pltpu.make_async_copy(ei_hbm.at[1, pl.ds(off, PC)], di, sem).wait()

    sets = [(siA_v, diA_v), (siB_v, diB_v)]
    _idx_load(0, *sets[0], semi)
    _idx_wait(0, *sets[0], semi)
    if PH > 1:
        _idx_load(1, *sets[1], semi)
    _gather(sets[0][0], 0, rows0_v, sem0)

    for p in range(PH):
        si, di = sets[p % 2]

        @pl.loop(0, (PC - 2) // 2)
        def _edges(i):
            a = 2 * i
            _gather(si, a + 1, rows1_v, sem1)
            _drain_scatter(si, di, a, rows0_v, sem0)
            _gather(si, a + 2, rows0_v, sem0)
            _drain_scatter(si, di, a + 1, rows1_v, sem1)

        _gather(si, PC - 1, rows1_v, sem1)
        _drain_scatter(si, di, PC - 2, rows0_v, sem0)
        if p + 1 < PH:
            nsi, ndi = sets[(p + 1) % 2]
            _idx_wait(p + 1, nsi, ndi, semi)
            _gather(nsi, 0, rows0_v, sem0)
            _drain_scatter(si, di, PC - 1, rows1_v, sem1)
            if p + 2 < PH:
                _idx_load(p + 2, si, di, semi)
        else:
            _drain_scatter(si, di, PC - 1, rows1_v, sem1)

    plsc.subcore_barrier()
    pltpu.sync_copy(acc_sh.at[pl.ds(s * RPT, RPT)],
                    out_hbm.at[c, pl.ds(s * RPT, RPT)])


def _dis_column(degr_ref):
    deg = degr_ref[0] + degr_ref[1] + 1.0
    return jnp.reshape(lax.rsqrt(deg), (NP, 1))[:N]


def _mm_body(x_ref, w_ref, degr_ref, y_ref):
    y_ref[...] = jnp.dot(x_ref[...] * _dis_column(degr_ref), w_ref[...],
                         preferred_element_type=jnp.float32)


def _comb_body(acc_ref, y_ref, degr_ref, b_ref, o_ref):
    o_ref[...] = _dis_column(degr_ref) * (
        acc_ref[0, :N] + acc_ref[1, :N] + y_ref[...]) + b_ref[...]


EROWS = E // K
PROWS = NW * CPW - EROWS


def kernel(x, edge_index, W, b):
    ei3 = edge_index.reshape(2, EROWS, K)
    ar = jnp.arange(PROWS * K, dtype=jnp.int32).reshape(PROWS, K)
    pads = jnp.stack([ar % 1024, N + ar % (NP - N)])
    eip = jnp.concatenate([ei3, pads], axis=1)

    degparts = _deg_kernel(eip)

    y = pl.pallas_call(
        _mm_body,
        out_shape=jax.ShapeDtypeStruct((N, D), jnp.float32),
    )(x, W, degparts)

    accparts = _edge_kernel(y, eip)

    out = pl.pallas_call(
        _comb_body,
        out_shape=jax.ShapeDtypeStruct((N, D), jnp.float32),
    )(accparts, y, degparts, b.reshape(1, D))

    return out


# speedup vs baseline: 50.8620x; 1.0067x over previous
import functools

import jax
import jax.numpy as jnp
from jax import lax
from jax.experimental import pallas as pl
from jax.experimental.pallas import tpu as pltpu
from jax.experimental.pallas import tpu_sc as plsc

N = 10000
E = 320000
D = 128
NP = 10240
NC, NS = 2, 16
NW = NC * NS
EPW = E // NW
K = 128
CPW = 80
PAD = CPW * K - EPW
PH = 5
PC = CPW // PH
RPT = NP // NS

_mesh = plsc.VectorSubcoreMesh(
    core_axis_name="c", subcore_axis_name="s", num_cores=NC, num_subcores=NS)


@functools.partial(
    pl.kernel,
    out_type=jax.ShapeDtypeStruct((NC, NP), jnp.float32),
    mesh=_mesh,
    scratch_types=[
        pltpu.VMEM((CPW, K), jnp.int32),
        pltpu.VMEM((K,), jnp.float32),
        pltpu.VMEM((RPT,), jnp.float32),
        pltpu.VMEM_SHARED((NP,), jnp.float32),
    ],
)
def _deg_kernel(ei_hbm, out_hbm, idx_v, ones_v, zb_v, deg_sh):
    c = lax.axis_index("c")
    s = lax.axis_index("s")
    wid = s * NC + c
    pltpu.sync_copy(ei_hbm.at[1, pl.ds(wid * CPW, CPW)], idx_v)
    for k in range(K // 16):
        ones_v[pl.ds(k * 16, 16)] = jnp.ones((16,), jnp.float32)

    @pl.loop(0, RPT // 16)
    def _zero(i):
        zb_v[pl.ds(i * 16, 16)] = jnp.zeros((16,), jnp.float32)

    pltpu.sync_copy(zb_v, deg_sh.at[pl.ds(s * RPT, RPT)])
    plsc.subcore_barrier()

    @pl.loop(0, CPW)
    def _hist(j):
        pltpu.sync_copy(ones_v, deg_sh.at[idx_v.at[j]], add=True)

    plsc.subcore_barrier()
    pltpu.sync_copy(deg_sh.at[pl.ds(s * RPT, RPT)],
                    out_hbm.at[c, pl.ds(s * RPT, RPT)])


@functools.partial(
    pl.kernel,
    out_type=jax.ShapeDtypeStruct((NC, NP, D), jnp.float32),
    mesh=_mesh,
    scratch_types=[
        pltpu.VMEM((PC, K), jnp.int32),
        pltpu.VMEM((PC, K), jnp.int32),
        pltpu.VMEM((PC, K), jnp.int32),
        pltpu.VMEM((PC, K), jnp.int32),
        pltpu.VMEM((K, D), jnp.float32),
        pltpu.VMEM((K, D), jnp.float32),
        pltpu.VMEM_SHARED((NP, D), jnp.float32),
        pltpu.SemaphoreType.DMA,
        pltpu.SemaphoreType.DMA,
        pltpu.SemaphoreType.DMA,
    ],
)
def _edge_kernel(y_hbm, ei_hbm, out_hbm,
                 siA_v, diA_v, siB_v, diB_v, rows0_v, rows1_v, acc_sh,
                 sem0, sem1, semi):
    c = lax.axis_index("c")
    s = lax.axis_index("s")
    wid = s * NC + c

    def _gather(si, j, buf, sem):
        pltpu.async_copy(y_hbm.at[si.at[j]], buf, sem)

    def _drain_scatter(si, di, j, buf, sem):
        pltpu.make_async_copy(y_hbm.at[si.at[j]], buf, sem).wait()
        pltpu.sync_copy(buf, acc_sh.at[di.at[j]], add=True)

    def _idx_load(p, si, di, sem):
        off = wid * CPW + p * PC
        pltpu.async_copy(ei_hbm.at[0, pl.ds(off, PC)], si, sem)
        pltpu.async_copy(ei_hbm.at[1, pl.ds(off, PC)], di, sem)

    def _idx_wait(p, si, di, sem):
        off = wid * CPW + p * PC
        pltpu.make_async_copy(ei_hbm.at[0, pl.ds(off, PC)], si, sem).wait()
        pltpu.make_async_copy(ei_hbm.at[1, pl.ds(off, PC)], di, sem).wait()

    sets = [(siA_v, diA_v), (siB_v, diB_v)]
    _idx_load(0, *sets[0], semi)

    @pl.loop(0, K)
    def _zero(r):
        for l in range(D // 16):
            rows0_v[r, pl.ds(l * 16, 16)] = jnp.zeros((16,), jnp.float32)

    for t in range(RPT // K):
        pltpu.async_copy(rows0_v, acc_sh.at[pl.ds(s * RPT + t * K, K)], sem0)
    for t in range(RPT // K):
        pltpu.make_async_copy(rows0_v, acc_sh.at[pl.ds(s * RPT + t * K, K)],
                              sem0).wait()
    plsc.subcore_barrier()

    _idx_wait(0, *sets[0], semi)
    if PH > 1:
        _idx_load(1, *sets[1], semi)
    _gather(sets[0][0], 0, rows0_v, sem0)

    for p in range(PH):
        si, di = sets[p % 2]

        @pl.loop(0, (PC - 2) // 2)
        def _edges(i):
            a = 2 * i
            _gather(si, a + 1, rows1_v, sem1)
            _drain_scatter(si, di, a, rows0_v, sem0)
            _gather(si, a + 2, rows0_v, sem0)
            _drain_scatter(si, di, a + 1, rows1_v, sem1)

        _gather(si, PC - 1, rows1_v, sem1)
        _drain_scatter(si, di, PC - 2, rows0_v, sem0)
        if p + 1 < PH:
            nsi, ndi = sets[(p + 1) % 2]
            _idx_wait(p + 1, nsi, ndi, semi)
            _gather(nsi, 0, rows0_v, sem0)
            _drain_scatter(si, di, PC - 1, rows1_v, sem1)
            if p + 2 < PH:
                _idx_load(p + 2, si, di, semi)
        else:
            _drain_scatter(si, di, PC - 1, rows1_v, sem1)

    plsc.subcore_barrier()
    pltpu.sync_copy(acc_sh.at[pl.ds(s * RPT, RPT)],
                    out_hbm.at[c, pl.ds(s * RPT, RPT)])


def _dis_column(degr_ref):
    deg = degr_ref[0] + degr_ref[1] + 1.0
    return jnp.reshape(lax.rsqrt(deg), (NP, 1))[:N]


def _mm_body(x_ref, w_ref, degr_ref, y_ref):
    y_ref[...] = jnp.dot(x_ref[...] * _dis_column(degr_ref), w_ref[...],
                         preferred_element_type=jnp.float32)


def _comb_body(acc_ref, y_ref, degr_ref, b_ref, o_ref):
    o_ref[...] = _dis_column(degr_ref) * (
        acc_ref[0, :N] + acc_ref[1, :N] + y_ref[...]) + b_ref[...]


EROWS = E // K
PROWS = NW * CPW - EROWS


def kernel(x, edge_index, W, b):
    ei3 = edge_index.reshape(2, EROWS, K)
    ar = jnp.arange(PROWS * K, dtype=jnp.int32).reshape(PROWS, K)
    pads = jnp.stack([ar % 1024, N + ar % (NP - N)])
    eip = jnp.concatenate([ei3, pads], axis=1)

    degparts = _deg_kernel(eip)

    y = pl.pallas_call(
        _mm_body,
        out_shape=jax.ShapeDtypeStruct((N, D), jnp.float32),
    )(x, W, degparts)

    accparts = _edge_kernel(y, eip)

    out = pl.pallas_call(
        _comb_body,
        out_shape=jax.ShapeDtypeStruct((N, D), jnp.float32),
    )(accparts, y, degparts, b.reshape(1, D))

    return out


# speedup vs baseline: 51.0225x; 1.0032x over previous
import functools

import jax
import jax.numpy as jnp
from jax import lax
from jax.experimental import pallas as pl
from jax.experimental.pallas import tpu as pltpu
from jax.experimental.pallas import tpu_sc as plsc

N = 10000
E = 320000
D = 128
NP = 10240
NC, NS = 2, 16
NW = NC * NS
K = 128
CPW = 80
PH = 5
PC = CPW // PH
RPT = NP // NS

_mesh = plsc.VectorSubcoreMesh(
    core_axis_name="c", subcore_axis_name="s", num_cores=NC, num_subcores=NS)


@functools.partial(
    pl.kernel,
    out_type=jax.ShapeDtypeStruct((NC, NP), jnp.float32),
    mesh=_mesh,
    scratch_types=[
        pltpu.VMEM((CPW, K), jnp.int32),
        pltpu.VMEM((K,), jnp.float32),
        pltpu.VMEM((RPT,), jnp.float32),
        pltpu.VMEM_SHARED((NP,), jnp.float32),
    ],
)
def _deg_kernel(ei_hbm, out_hbm, idx_v, ones_v, zb_v, deg_sh):
    c = lax.axis_index("c")
    s = lax.axis_index("s")
    wid = s * NC + c
    pltpu.sync_copy(ei_hbm.at[1, pl.ds(wid * CPW, CPW)], idx_v)
    for k in range(K // 16):
        ones_v[pl.ds(k * 16, 16)] = jnp.ones((16,), jnp.float32)

    @pl.loop(0, RPT // 16)
    def _zero(i):
        zb_v[pl.ds(i * 16, 16)] = jnp.zeros((16,), jnp.float32)

    pltpu.sync_copy(zb_v, deg_sh.at[pl.ds(s * RPT, RPT)])
    plsc.subcore_barrier()

    @pl.loop(0, CPW)
    def _hist(j):
        pltpu.sync_copy(ones_v, deg_sh.at[idx_v.at[j]], add=True)

    plsc.subcore_barrier()
    pltpu.sync_copy(deg_sh.at[pl.ds(s * RPT, RPT)],
                    out_hbm.at[c, pl.ds(s * RPT, RPT)])


@functools.partial(
    pl.kernel,
    out_type=jax.ShapeDtypeStruct((NC, NP, D), jnp.float32),
    mesh=_mesh,
    scratch_types=[
        pltpu.VMEM((PC, K), jnp.int32),
        pltpu.VMEM((PC, K), jnp.int32),
        pltpu.VMEM((PC, K), jnp.int32),
        pltpu.VMEM((PC, K), jnp.int32),
        pltpu.VMEM((K, D), jnp.float32),
        pltpu.VMEM((K, D), jnp.float32),
        pltpu.VMEM_SHARED((NP, D), jnp.float32),
        pltpu.SemaphoreType.DMA,
        pltpu.SemaphoreType.DMA,
        pltpu.SemaphoreType.DMA,
    ],
)
def _edge_kernel(y_hbm, ei_hbm, out_hbm,
                 siA_v, diA_v, siB_v, diB_v, rows0_v, rows1_v, acc_sh,
                 sem0, sem1, semi):
    c = lax.axis_index("c")
    s = lax.axis_index("s")
    wid = s * NC + c

    def _gather(si, j, buf, sem):
        pltpu.async_copy(y_hbm.at[si.at[j]], buf, sem)

    def _drain_scatter(si, di, j, buf, sem):
        pltpu.make_async_copy(y_hbm.at[si.at[j]], buf, sem).wait()
        pltpu.sync_copy(buf, acc_sh.at[di.at[j]], add=True)

    def _idx_load(p, si, di, sem):
        off = wid * CPW + p * PC
        pltpu.async_copy(ei_hbm.at[0, pl.ds(off, PC)], si, sem)
        pltpu.async_copy(ei_hbm.at[1, pl.ds(off, PC)], di, sem)

    def _idx_wait(p, si, di, sem):
        off = wid * CPW + p * PC
        pltpu.make_async_copy(ei_hbm.at[0, pl.ds(off, PC)], si, sem).wait()
        pltpu.make_async_copy(ei_hbm.at[1, pl.ds(off, PC)], di, sem).wait()

    sets = [(siA_v, diA_v), (siB_v, diB_v)]
    _idx_load(0, *sets[0], semi)

    @pl.loop(0, K)
    def _zero(r):
        for l in range(D // 16):
            rows0_v[r, pl.ds(l * 16, 16)] = jnp.zeros((16,), jnp.float32)

    for t in range(RPT // K):
        pltpu.async_copy(rows0_v, acc_sh.at[pl.ds(s * RPT + t * K, K)], sem0)
    for t in range(RPT // K):
        pltpu.make_async_copy(rows0_v, acc_sh.at[pl.ds(s * RPT + t * K, K)],
                              sem0).wait()
    plsc.subcore_barrier()

    _idx_wait(0, *sets[0], semi)
    if PH > 1:
        _idx_load(1, *sets[1], semi)
    _gather(sets[0][0], 0, rows0_v, sem0)

    for p in range(PH):
        si, di = sets[p % 2]

        @pl.loop(0, (PC - 2) // 2)
        def _edges(i):
            a = 2 * i
            _gather(si, a + 1, rows1_v, sem1)
            _drain_scatter(si, di, a, rows0_v, sem0)
            _gather(si, a + 2, rows0_v, sem0)
            _drain_scatter(si, di, a + 1, rows1_v, sem1)

        _gather(si, PC - 1, rows1_v, sem1)
        _drain_scatter(si, di, PC - 2, rows0_v, sem0)
        if p + 1 < PH:
            nsi, ndi = sets[(p + 1) % 2]
            _idx_wait(p + 1, nsi, ndi, semi)
            _gather(nsi, 0, rows0_v, sem0)
            _drain_scatter(si, di, PC - 1, rows1_v, sem1)
            if p + 2 < PH:
                _idx_load(p + 2, si, di, semi)
        else:
            _drain_scatter(si, di, PC - 1, rows1_v, sem1)

    plsc.subcore_barrier()
    pltpu.sync_copy(acc_sh.at[pl.ds(s * RPT, RPT)],
                    out_hbm.at[c, pl.ds(s * RPT, RPT)])


def _dis_column(degr_ref):
    deg = degr_ref[0] + degr_ref[1] + 1.0
    return jnp.reshape(lax.rsqrt(deg), (NP, 1))[:N]


def _mm_body(x_ref, w_ref, degr_ref, y_ref):
    y_ref[...] = jnp.dot(x_ref[...] * _dis_column(degr_ref), w_ref[...],
                         preferred_element_type=jnp.float32)


def _comb_body(acc_ref, y_ref, degr_ref, b_ref, o_ref):
    o_ref[...] = _dis_column(degr_ref) * (
        acc_ref[0, :N] + acc_ref[1, :N] + y_ref[...]) + b_ref[...]


EROWS = E // K
PROWS = NW * CPW - EROWS


def kernel(x, edge_index, W, b):
    ei3 = edge_index.reshape(2, EROWS, K)
    ar = jnp.arange(PROWS * K, dtype=jnp.int32).reshape(PROWS, K)
    pads = jnp.stack([ar % 1024, N + ar % (NP - N)])
    eip = jnp.concatenate([ei3, pads], axis=1)

    degparts = _deg_kernel(eip)

    y = pl.pallas_call(
        _mm_body,
        out_shape=jax.ShapeDtypeStruct((N, D), jnp.float32),
    )(x, W, degparts)

    accparts = _edge_kernel(y, eip)

    out = pl.pallas_call(
        _comb_body,
        out_shape=jax.ShapeDtypeStruct((N, D), jnp.float32),
    )(accparts, y, degparts, b.reshape(1, D))

    return out
